# baseline (speedup 1.0000x reference)
import functools

import jax
import jax.numpy as jnp
from jax import lax
from jax.experimental import pallas as pl
from jax.experimental.pallas import tpu as pltpu
from jax.experimental.pallas import tpu_sc as plsc

DIM = 1024
MAX_SEQ_LEN = 8192
SCALE = DIM ** (-0.5)

NUM_CORES = 2
NUM_SUBCORES = 16
NW = NUM_CORES * NUM_SUBCORES
ROWS_W = MAX_SEQ_LEN // NW
CROWS = 16
NCHUNK = ROWS_W // CROWS
NGBUF = 3
NSBUF = 4
LANES = 16
CGROUPS = DIM // LANES

_mesh = plsc.VectorSubcoreMesh(core_axis_name="c", subcore_axis_name="s")


@functools.partial(
    pl.kernel,
    mesh=_mesh,
    out_type=jax.ShapeDtypeStruct((MAX_SEQ_LEN, DIM), jnp.float32),
    scratch_types=(
        [pltpu.VMEM((CROWS, DIM), jnp.float32)] * (NGBUF + NSBUF)
        + [pltpu.SemaphoreType.DMA] * (NGBUF + NSBUF)
    ),
    compiler_params=pltpu.CompilerParams(use_tc_tiling_on_sc=True, disable_bounds_checks=True, disable_semaphore_checks=True, skip_device_barrier=True),
)
def _scale_copy(emb_hbm, out_hbm, *refs):
    gbufs = refs[:NGBUF]
    sbufs = refs[NGBUF:NGBUF + NSBUF]
    gsems = refs[NGBUF + NSBUF:2 * NGBUF + NSBUF]
    ssems = refs[2 * NGBUF + NSBUF:]
    wid = lax.axis_index("s") * NUM_CORES + lax.axis_index("c")
    base = wid * ROWS_W

    def gather(c):
        src = emb_hbm.at[pl.ds(base + c * CROWS, CROWS), :]
        return pltpu.async_copy(src, gbufs[c % NGBUF], gsems[c % NGBUF])

    def scatter(c):
        dst = out_hbm.at[pl.ds(base + c * CROWS, CROWS), :]
        return pltpu.async_copy(sbufs[c % NSBUF], dst, ssems[c % NSBUF])

    gh = [None] * NCHUNK
    sh = [None] * NCHUNK
    for k in range(NGBUF):
        gh[k] = gather(k)
    for c in range(NCHUNK):
        gh[c].wait()
        if c >= NSBUF:
            sh[c - NSBUF].wait()
        gbuf = gbufs[c % NGBUF]
        sbuf = sbufs[c % NSBUF]

        @plsc.parallel_loop(0, CROWS * CGROUPS, unroll=8)
        def _mul(i):
            r = i >> 6
            sl = pl.ds((i & (CGROUPS - 1)) * LANES, LANES)
            sbuf[r, sl] = gbuf[r, sl] * SCALE

        sh[c] = scatter(c)
        if c + NGBUF < NCHUNK:
            gh[c + NGBUF] = gather(c + NGBUF)
    for c in range(NCHUNK - NSBUF, NCHUNK):
        sh[c].wait()


def kernel(x, emb):
    seq_len = x.shape[1]
    return _scale_copy(emb)[:seq_len]


# speedup vs baseline: 1.0525x; 1.0525x over previous
import functools

import jax
import jax.numpy as jnp
from jax import lax
from jax.experimental import pallas as pl
from jax.experimental.pallas import tpu as pltpu
from jax.experimental.pallas import tpu_sc as plsc

DIM = 1024
MAX_SEQ_LEN = 8192
SCALE = DIM ** (-0.5)

NUM_CORES = 2
NUM_SUBCORES = 16
NW = NUM_CORES * NUM_SUBCORES
ROWS_W = MAX_SEQ_LEN // NW
CROWS = 16
NCHUNK = ROWS_W // CROWS
RING = 4
LANES = 16
CGROUPS = DIM // LANES

_mesh = plsc.VectorSubcoreMesh(core_axis_name="c", subcore_axis_name="s")


@functools.partial(
    pl.kernel,
    mesh=_mesh,
    out_type=jax.ShapeDtypeStruct((MAX_SEQ_LEN, DIM), jnp.float32),
    scratch_types=[
        pltpu.VMEM((RING * CROWS, DIM), jnp.float32),
        pltpu.VMEM((RING * CROWS, DIM), jnp.float32),
        pltpu.SemaphoreType.DMA((RING,)),
        pltpu.SemaphoreType.DMA((RING,)),
    ],
    compiler_params=pltpu.CompilerParams(use_tc_tiling_on_sc=True),
)
def _scale_copy(emb_hbm, out_hbm, gbuf, sbuf, gsem, ssem):
    wid = lax.axis_index("s") * NUM_CORES + lax.axis_index("c")
    base = wid * ROWS_W

    def rows(c):
        return pl.ds(base + c * CROWS, CROWS)

    def slot(c):
        return lax.rem(c, RING)

    def gather_start(c):
        b = slot(c)
        pltpu.async_copy(
            emb_hbm.at[rows(c), :],
            gbuf.at[pl.ds(b * CROWS, CROWS), :],
            gsem.at[b],
        )

    def gather_wait(c):
        b = slot(c)
        pltpu.make_async_copy(
            emb_hbm.at[rows(c), :],
            gbuf.at[pl.ds(b * CROWS, CROWS), :],
            gsem.at[b],
        ).wait()

    def scatter_start(c):
        b = slot(c)
        pltpu.async_copy(
            sbuf.at[pl.ds(b * CROWS, CROWS), :],
            out_hbm.at[rows(c), :],
            ssem.at[b],
        )

    def scatter_wait(c):
        b = slot(c)
        pltpu.make_async_copy(
            sbuf.at[pl.ds(b * CROWS, CROWS), :],
            out_hbm.at[rows(c), :],
            ssem.at[b],
        ).wait()

    for k in range(RING):
        gather_start(k)

    def chunk_body(c, carry):
        gather_wait(c)

        @pl.when(c >= RING)
        def _():
            scatter_wait(c - RING)

        b = slot(c)
        goff = b * CROWS
        soff = b * CROWS

        @plsc.parallel_loop(0, CROWS * CGROUPS, unroll=8)
        def _mul(i):
            r = i >> 6
            sl = pl.ds((i & (CGROUPS - 1)) * LANES, LANES)
            sbuf[soff + r, sl] = gbuf[goff + r, sl] * SCALE

        scatter_start(c)

        @pl.when(c + RING < NCHUNK)
        def _():
            gather_start(c + RING)

        return carry

    lax.fori_loop(0, NCHUNK, chunk_body, 0)
    for k in range(NCHUNK - RING, NCHUNK):
        scatter_wait(k)


def kernel(x, emb):
    seq_len = x.shape[1]
    return _scale_copy(emb)[:seq_len]


# speedup vs baseline: 1.0813x; 1.0273x over previous
import functools

import jax
import jax.numpy as jnp
from jax import lax
from jax.experimental import pallas as pl
from jax.experimental.pallas import tpu as pltpu
from jax.experimental.pallas import tpu_sc as plsc

DIM = 1024
MAX_SEQ_LEN = 8192
SCALE = DIM ** (-0.5)

NUM_CORES = 2
NUM_SUBCORES = 16
NW = NUM_CORES * NUM_SUBCORES
ROWS_W = MAX_SEQ_LEN // NW
CROWS = 8
NCHUNK = ROWS_W // CROWS
RING = 8
LANES = 16
CGROUPS = DIM // LANES

_mesh = plsc.VectorSubcoreMesh(core_axis_name="c", subcore_axis_name="s")


@functools.partial(
    pl.kernel,
    mesh=_mesh,
    out_type=jax.ShapeDtypeStruct((MAX_SEQ_LEN, DIM), jnp.float32),
    scratch_types=[
        pltpu.VMEM((RING * CROWS, DIM), jnp.float32),
        pltpu.VMEM((RING * CROWS, DIM), jnp.float32),
        pltpu.SemaphoreType.DMA((RING,)),
        pltpu.SemaphoreType.DMA((RING,)),
    ],
    compiler_params=pltpu.CompilerParams(use_tc_tiling_on_sc=True),
)
def _scale_copy(emb_hbm, out_hbm, gbuf, sbuf, gsem, ssem):
    wid = lax.axis_index("s") * NUM_CORES + lax.axis_index("c")
    base = wid * ROWS_W

    def rows(c):
        return pl.ds(base + c * CROWS, CROWS)

    def slot(c):
        return lax.rem(c, RING)

    def gather_start(c):
        b = slot(c)
        pltpu.async_copy(
            emb_hbm.at[rows(c), :],
            gbuf.at[pl.ds(b * CROWS, CROWS), :],
            gsem.at[b],
        )

    def gather_wait(c):
        b = slot(c)
        pltpu.make_async_copy(
            emb_hbm.at[rows(c), :],
            gbuf.at[pl.ds(b * CROWS, CROWS), :],
            gsem.at[b],
        ).wait()

    def scatter_start(c):
        b = slot(c)
        pltpu.async_copy(
            sbuf.at[pl.ds(b * CROWS, CROWS), :],
            out_hbm.at[rows(c), :],
            ssem.at[b],
        )

    def scatter_wait(c):
        b = slot(c)
        pltpu.make_async_copy(
            sbuf.at[pl.ds(b * CROWS, CROWS), :],
            out_hbm.at[rows(c), :],
            ssem.at[b],
        ).wait()

    for k in range(RING):
        gather_start(k)

    def chunk_body(c, carry):
        gather_wait(c)

        @pl.when(c >= RING)
        def _():
            scatter_wait(c - RING)

        b = slot(c)
        goff = b * CROWS
        soff = b * CROWS

        @plsc.parallel_loop(0, CROWS * CGROUPS, unroll=8)
        def _mul(i):
            r = i >> 6
            sl = pl.ds((i & (CGROUPS - 1)) * LANES, LANES)
            sbuf[soff + r, sl] = gbuf[goff + r, sl] * SCALE

        scatter_start(c)

        @pl.when(c + RING < NCHUNK)
        def _():
            gather_start(c + RING)

        return carry

    lax.fori_loop(0, NCHUNK, chunk_body, 0)
    for k in range(NCHUNK - RING, NCHUNK):
        scatter_wait(k)


def kernel(x, emb):
    seq_len = x.shape[1]
    return _scale_copy(emb)[:seq_len]
